# baseline (device time: 125610 ns/iter reference)
import jax
import jax.numpy as jnp
from jax import lax
from jax.experimental import pallas as pl
from jax.experimental.pallas import tpu as pltpu

N_DEV = 4
B = 256
B_SH = B // N_DEV
D = 2048
H_SH = 4096
T = 8
HT = H_SH // T
N_TILES = 3 * T
N_SLOTS = 4


def kernel(x, Win0, Wout0, Win1, Wout1, Win2, Wout2):
    def body(
        x_ref, win0, wout0, win1, wout1, win2, wout2, out_ref,
        x_full, part, rs_snd_buf, rs_rcv_buf, win_vmem, win_bf, h_buf,
        wout_vmem,
        ag_snd, ag_rcv, rs_snd, rs_rcv, win_sem, wout_sem,
    ):
        p = lax.axis_index("i")
        peers = [lax.rem(p + off, N_DEV) for off in (1, 2, 3)]

        wins = (win0, win1, win2)
        wouts = (wout0, wout1, wout2)

        def issue_win(k):
            if k < N_TILES:
                l, t = divmod(k, T)
                pltpu.make_async_copy(
                    wins[l].at[:, pl.ds(t * HT, HT)],
                    win_vmem.at[k % 2], win_sem.at[k % 2],
                ).start()

        def issue_wout(k):
            if k < N_TILES:
                l, t = divmod(k, T)
                pltpu.make_async_copy(
                    wouts[l].at[pl.ds(t * HT, HT), :],
                    wout_vmem.at[k % N_SLOTS], wout_sem.at[k % N_SLOTS],
                ).start()

        for k in range(2):
            issue_win(k)
        for k in range(N_SLOTS):
            issue_wout(k)

        barrier_sem = pltpu.get_barrier_semaphore()
        for q in peers:
            pl.semaphore_signal(
                barrier_sem, inc=1,
                device_id=(q,), device_id_type=pl.DeviceIdType.MESH,
            )
        pl.semaphore_wait(barrier_sem, 3)

        def my_rows(ref):
            return ref.at[pl.ds(p * B_SH, B_SH), :]

        def ag_start():
            sends = []
            for off in (1, 2, 3):
                q = peers[off - 1]
                s = pltpu.make_async_remote_copy(
                    src_ref=my_rows(x_full),
                    dst_ref=my_rows(x_full),
                    send_sem=ag_snd.at[off - 1],
                    recv_sem=ag_rcv.at[p],
                    device_id=(q,),
                    device_id_type=pl.DeviceIdType.MESH,
                )
                s.start()
                sends.append(s)
            return sends

        def ag_wait_recv(off):
            q = peers[off - 1]
            r = pltpu.make_async_remote_copy(
                src_ref=my_rows(x_full),
                dst_ref=x_full.at[pl.ds(q * B_SH, B_SH), :],
                send_sem=ag_snd.at[off - 1],
                recv_sem=ag_rcv.at[q],
                device_id=(q,),
                device_id_type=pl.DeviceIdType.MESH,
            )
            r.wait_recv()
            return q

        x_full[pl.ds(p * B_SH, B_SH), :] = x_ref[:, :].astype(jnp.bfloat16)
        ag_sends = ag_start()

        for l in range(3):
            xc = x_full[pl.ds(p * B_SH, B_SH), :]
            for t in range(T):
                k = l * T + t
                slot = k % 2
                pltpu.make_async_copy(
                    wins[l].at[:, pl.ds(t * HT, HT)],
                    win_vmem.at[slot], win_sem.at[slot],
                ).wait()
                wt = win_vmem[slot].astype(jnp.bfloat16)
                win_bf[:, pl.ds(t * HT, HT)] = wt
                issue_win(k + 2)
                h_buf[pl.ds(p * B_SH, B_SH), pl.ds(t * HT, HT)] = (
                    jnp.maximum(
                        jnp.dot(xc, wt, preferred_element_type=jnp.float32),
                        0.0,
                    ).astype(jnp.bfloat16)
                )
            for off in (1, 2, 3):
                q = ag_wait_recv(off)
                xq = x_full[pl.ds(q * B_SH, B_SH), :]
                h_buf[pl.ds(q * B_SH, B_SH), :] = jnp.maximum(
                    jnp.dot(
                        xq, win_bf[:, :],
                        preferred_element_type=jnp.float32,
                    ),
                    0.0,
                ).astype(jnp.bfloat16)
            for s in ag_sends:
                s.wait_send()

            acc = jnp.zeros((B, D), jnp.float32)
            for t in range(T):
                k = l * T + t
                slot = k % N_SLOTS
                pltpu.make_async_copy(
                    wouts[l].at[pl.ds(t * HT, HT), :],
                    wout_vmem.at[slot], wout_sem.at[slot],
                ).wait()
                acc = acc + jnp.dot(
                    h_buf[:, pl.ds(t * HT, HT)],
                    wout_vmem[slot].astype(jnp.bfloat16),
                    preferred_element_type=jnp.float32,
                )
                issue_wout(k + N_SLOTS)
            part[:, :] = acc

            sends = []
            for off in (1, 2, 3):
                q = peers[off - 1]
                rs_snd_buf[pl.ds((off - 1) * B_SH, B_SH), :] = (
                    part[pl.ds(q * B_SH, B_SH), :].astype(jnp.bfloat16)
                )
                s = pltpu.make_async_remote_copy(
                    src_ref=rs_snd_buf.at[pl.ds((off - 1) * B_SH, B_SH), :],
                    dst_ref=rs_rcv_buf.at[pl.ds(p * B_SH, B_SH), :],
                    send_sem=rs_snd.at[off - 1],
                    recv_sem=rs_rcv.at[p],
                    device_id=(q,),
                    device_id_type=pl.DeviceIdType.MESH,
                )
                s.start()
                sends.append(s)
            tot = part[pl.ds(p * B_SH, B_SH), :]
            for off in (1, 2, 3):
                q = peers[off - 1]
                r = pltpu.make_async_remote_copy(
                    src_ref=rs_snd_buf.at[pl.ds(0, B_SH), :],
                    dst_ref=rs_rcv_buf.at[pl.ds(q * B_SH, B_SH), :],
                    send_sem=rs_snd.at[off - 1],
                    recv_sem=rs_rcv.at[q],
                    device_id=(q,),
                    device_id_type=pl.DeviceIdType.MESH,
                )
                r.wait_recv()
                tot = tot + rs_rcv_buf[pl.ds(q * B_SH, B_SH), :].astype(
                    jnp.float32
                )
            for s in sends:
                s.wait_send()

            x_full[pl.ds(p * B_SH, B_SH), :] = tot.astype(jnp.bfloat16)
            ag_sends = ag_start()

        for off in (1, 2, 3):
            ag_wait_recv(off)
        for s in ag_sends:
            s.wait_send()
        out_ref[:, :] = x_full[:, :].astype(jnp.float32)

    hbm = pl.BlockSpec(memory_space=pltpu.MemorySpace.HBM)
    return pl.pallas_call(
        body,
        out_shape=jax.ShapeDtypeStruct((B, D), jnp.float32),
        in_specs=[pl.BlockSpec(memory_space=pltpu.VMEM)] + [hbm] * 6,
        out_specs=pl.BlockSpec(memory_space=pltpu.VMEM),
        scratch_shapes=[
            pltpu.VMEM((B, D), jnp.bfloat16),
            pltpu.VMEM((B, D), jnp.float32),
            pltpu.VMEM(((N_DEV - 1) * B_SH, D), jnp.bfloat16),
            pltpu.VMEM((B, D), jnp.bfloat16),
            pltpu.VMEM((2, D, HT), jnp.float32),
            pltpu.VMEM((D, H_SH), jnp.bfloat16),
            pltpu.VMEM((B, H_SH), jnp.bfloat16),
            pltpu.VMEM((N_SLOTS, HT, D), jnp.float32),
            pltpu.SemaphoreType.DMA((N_DEV - 1,)),
            pltpu.SemaphoreType.DMA((N_DEV,)),
            pltpu.SemaphoreType.DMA((N_DEV - 1,)),
            pltpu.SemaphoreType.DMA((N_DEV,)),
            pltpu.SemaphoreType.DMA((2,)),
            pltpu.SemaphoreType.DMA((N_SLOTS,)),
        ],
        compiler_params=pltpu.CompilerParams(
            collective_id=0, vmem_limit_bytes=60 * 1024 * 1024
        ),
    )(x, Win0, Wout0, Win1, Wout1, Win2, Wout2)


# device time: 105111 ns/iter; 1.1950x vs baseline; 1.1950x over previous
import jax
import jax.numpy as jnp
from jax import lax
from jax.experimental import pallas as pl
from jax.experimental.pallas import tpu as pltpu

N_DEV = 4
B = 256
B_SH = B // N_DEV
D = 2048
H_SH = 4096
T = 8
HT = H_SH // T
N_TILES = 3 * T
N_SLOTS = 4


def kernel(x, Win0, Wout0, Win1, Wout1, Win2, Wout2):
    def body(
        x_ref, win0, wout0, win1, wout1, win2, wout2, out_ref,
        x_full, part, rs_snd_buf, rs_rcv_buf, win_vmem, wout_vmem,
        ag_snd, ag_rcv, rs_snd, rs_rcv, win_sem, wout_sem,
    ):
        p = lax.axis_index("i")
        peers = [lax.rem(p + off, N_DEV) for off in (1, 2, 3)]

        wins = (win0, win1, win2)
        wouts = (wout0, wout1, wout2)

        def issue_win_layer(l):
            if l < 3:
                pltpu.make_async_copy(
                    wins[l], win_vmem, win_sem.at[0]
                ).start()

        def issue_wout(k):
            if k < N_TILES:
                l, t = divmod(k, T)
                pltpu.make_async_copy(
                    wouts[l].at[pl.ds(t * HT, HT), :],
                    wout_vmem.at[k % N_SLOTS], wout_sem.at[k % N_SLOTS],
                ).start()

        issue_win_layer(0)
        for k in range(N_SLOTS):
            issue_wout(k)

        barrier_sem = pltpu.get_barrier_semaphore()
        for q in peers:
            pl.semaphore_signal(
                barrier_sem, inc=1,
                device_id=(q,), device_id_type=pl.DeviceIdType.MESH,
            )
        pl.semaphore_wait(barrier_sem, 3)

        def my_rows(ref):
            return ref.at[pl.ds(p * B_SH, B_SH), :]

        def allgather_my_chunk():
            sends = []
            for off in (1, 2, 3):
                q = peers[off - 1]
                s = pltpu.make_async_remote_copy(
                    src_ref=my_rows(x_full),
                    dst_ref=my_rows(x_full),
                    send_sem=ag_snd.at[off - 1],
                    recv_sem=ag_rcv.at[p],
                    device_id=(q,),
                    device_id_type=pl.DeviceIdType.MESH,
                )
                s.start()
                sends.append(s)
            for off in (1, 2, 3):
                q = peers[off - 1]
                r = pltpu.make_async_remote_copy(
                    src_ref=my_rows(x_full),
                    dst_ref=x_full.at[pl.ds(q * B_SH, B_SH), :],
                    send_sem=ag_snd.at[off - 1],
                    recv_sem=ag_rcv.at[q],
                    device_id=(q,),
                    device_id_type=pl.DeviceIdType.MESH,
                )
                r.wait_recv()
            for s in sends:
                s.wait_send()

        x_full[pl.ds(p * B_SH, B_SH), :] = x_ref[:, :].astype(jnp.bfloat16)
        allgather_my_chunk()

        for l in range(3):
            xf = x_full[:, :]
            acc = jnp.zeros((B, D), jnp.float32)
            pltpu.make_async_copy(wins[l], win_vmem, win_sem.at[0]).wait()
            for t in range(T):
                k = l * T + t
                slot = k % N_SLOTS
                h_t = jnp.maximum(
                    jnp.dot(
                        xf,
                        win_vmem[:, t * HT:(t + 1) * HT].astype(jnp.bfloat16),
                        preferred_element_type=jnp.float32,
                    ),
                    0.0,
                ).astype(jnp.bfloat16)
                if t == T - 1:
                    issue_win_layer(l + 1)
                pltpu.make_async_copy(
                    wouts[l].at[pl.ds(t * HT, HT), :],
                    wout_vmem.at[slot], wout_sem.at[slot],
                ).wait()
                acc = acc + jnp.dot(
                    h_t, wout_vmem[slot].astype(jnp.bfloat16),
                    preferred_element_type=jnp.float32,
                )
                issue_wout(k + N_SLOTS)
            part[:, :] = acc

            sends = []
            for off in (1, 2, 3):
                q = peers[off - 1]
                rs_snd_buf[pl.ds((off - 1) * B_SH, B_SH), :] = (
                    part[pl.ds(q * B_SH, B_SH), :].astype(jnp.bfloat16)
                )
                s = pltpu.make_async_remote_copy(
                    src_ref=rs_snd_buf.at[pl.ds((off - 1) * B_SH, B_SH), :],
                    dst_ref=rs_rcv_buf.at[pl.ds(p * B_SH, B_SH), :],
                    send_sem=rs_snd.at[off - 1],
                    recv_sem=rs_rcv.at[p],
                    device_id=(q,),
                    device_id_type=pl.DeviceIdType.MESH,
                )
                s.start()
                sends.append(s)
            tot = part[pl.ds(p * B_SH, B_SH), :]
            for off in (1, 2, 3):
                q = peers[off - 1]
                r = pltpu.make_async_remote_copy(
                    src_ref=rs_snd_buf.at[pl.ds(0, B_SH), :],
                    dst_ref=rs_rcv_buf.at[pl.ds(q * B_SH, B_SH), :],
                    send_sem=rs_snd.at[off - 1],
                    recv_sem=rs_rcv.at[q],
                    device_id=(q,),
                    device_id_type=pl.DeviceIdType.MESH,
                )
                r.wait_recv()
                tot = tot + rs_rcv_buf[pl.ds(q * B_SH, B_SH), :].astype(
                    jnp.float32
                )
            for s in sends:
                s.wait_send()

            x_full[pl.ds(p * B_SH, B_SH), :] = tot.astype(jnp.bfloat16)
            allgather_my_chunk()

        out_ref[:, :] = x_full[:, :].astype(jnp.float32)

    hbm = pl.BlockSpec(memory_space=pltpu.MemorySpace.HBM)
    return pl.pallas_call(
        body,
        out_shape=jax.ShapeDtypeStruct((B, D), jnp.float32),
        in_specs=[pl.BlockSpec(memory_space=pltpu.VMEM)] + [hbm] * 6,
        out_specs=pl.BlockSpec(memory_space=pltpu.VMEM),
        scratch_shapes=[
            pltpu.VMEM((B, D), jnp.bfloat16),
            pltpu.VMEM((B, D), jnp.float32),
            pltpu.VMEM(((N_DEV - 1) * B_SH, D), jnp.bfloat16),
            pltpu.VMEM((B, D), jnp.bfloat16),
            pltpu.VMEM((D, H_SH), jnp.float32),
            pltpu.VMEM((N_SLOTS, HT, D), jnp.float32),
            pltpu.SemaphoreType.DMA((N_DEV - 1,)),
            pltpu.SemaphoreType.DMA((N_DEV,)),
            pltpu.SemaphoreType.DMA((N_DEV - 1,)),
            pltpu.SemaphoreType.DMA((N_DEV,)),
            pltpu.SemaphoreType.DMA((1,)),
            pltpu.SemaphoreType.DMA((N_SLOTS,)),
        ],
        compiler_params=pltpu.CompilerParams(
            collective_id=0, vmem_limit_bytes=63 * 1024 * 1024
        ),
    )(x, Win0, Wout0, Win1, Wout1, Win2, Wout2)
